# baseline (speedup 1.0000x reference)
import jax
import jax.numpy as jnp
from jax import lax
from jax.experimental import pallas as pl
from jax.experimental.pallas import tpu as pltpu
from jax.experimental.pallas import tpu_sc as plsc

D = 128
N_NODES = 10000
N_EDGES = 320000

NC = 2
NS = 16
NW = NC * NS

NPAD = 10240
ROWS_PT = NPAD // NS

CH = 80
NCH = 252
EPT = NCH * CH
E_TOT = NW * EPT


def _lane_bcast(vec, j):
    idx = jnp.full((16, 1), j, dtype=jnp.int32)
    return lax.gather(
        vec, idx,
        lax.GatherDimensionNumbers(
            offset_dims=(), collapsed_slice_dims=(0,), start_index_map=(0,)),
        (1,),
        mode=lax.GatherScatterMode.PROMISE_IN_BOUNDS)


def _sc_body(tab, srch, dsth, wh, zeros_hbm, out,
             acc, rows0, rows1, rows2, rows3,
             srcs0, srcs1, srcs2, srcs3, dsts0, dsts1, dsts2, dsts3,
             ws0, ws1, ws2, ws3, dsc0, dsc1, g0, g1, s0, s1, isem):
    cid = lax.axis_index("c")
    sid = lax.axis_index("s")
    wid = sid * NC + cid

    rows = (rows0, rows1, rows2, rows3)
    srcs = (srcs0, srcs1, srcs2, srcs3)
    dsts = (dsts0, dsts1, dsts2, dsts3)
    ws = (ws0, ws1, ws2, ws3)
    dsc = (dsc0, dsc1)
    gsem = (g0, g1)
    ssem = (s0, s1)

    pltpu.sync_copy(zeros_hbm.at[pl.ds(sid * ROWS_PT, ROWS_PT)],
                    acc.at[pl.ds(sid * ROWS_PT, ROWS_PT)])
    plsc.subcore_barrier()

    base = wid * EPT

    def stage_idx(c, s):
        off = base + c * CH
        pltpu.async_copy(srch.at[pl.ds(off, CH)], srcs[s], isem)
        pltpu.async_copy(dsth.at[pl.ds(off, CH)], dsts[s], isem)
        pltpu.async_copy(wh.at[pl.ds(off, CH)], ws[s], isem)

    def wait_idx(s):
        pltpu.make_async_copy(srch.at[pl.ds(0, CH)], srcs[s], isem).wait()
        pltpu.make_async_copy(dsth.at[pl.ds(0, CH)], dsts[s], isem).wait()
        pltpu.make_async_copy(wh.at[pl.ds(0, CH)], ws[s], isem).wait()

    def start_gather(l):
        pltpu.async_copy(tab.at[srcs[l]], rows[l], gsem[l % 2])

    def wait_gather(l):
        pltpu.make_async_copy(tab.at[srcs[l]], rows[l], gsem[l % 2]).wait()

    def start_scatter(l, p):
        for k in range(CH // 16):
            dsc[p][pl.ds(16 * k, 16)] = dsts[l][pl.ds(16 * k, 16)]
        pltpu.async_copy(rows[l], acc.at[dsc[p]], ssem[p], add=True)

    def wait_scatter(l, p):
        pltpu.make_async_copy(rows[l], acc.at[dsc[p]], ssem[p]).wait()

    def scale(b, s):
        rbuf = rows[b]
        wref = ws[s]

        def grp(g, carry):
            wv = wref[pl.ds(16 * g, 16)]
            for j in range(16):
                wbc = _lane_bcast(wv, j)
                e = 16 * g + j
                for v in range(8):
                    rbuf[e, pl.ds(16 * v, 16)] = rbuf[e, pl.ds(16 * v, 16)] * wbc
            return carry
        lax.fori_loop(0, CH // 16, grp, 0)

    def chunk(c, r, first, guard):
        l = r
        l2 = (r + 2) % 4
        l3 = (r + 3) % 4
        wait_gather(l)
        if not first:
            wait_scatter(l2, r % 2)
        if guard:
            @pl.when(c + 2 < NCH)
            def _():
                wait_idx(l2)

            @pl.when(c + 3 < NCH)
            def _():
                stage_idx(c + 3, l3)

            @pl.when(c + 2 < NCH)
            def _():
                start_gather(l2)
        else:
            wait_idx(l2)
            stage_idx(c + 3, l3)
            start_gather(l2)
        scale(l, l)
        start_scatter(l, r % 2)

    stage_idx(0, 0)
    wait_idx(0)
    start_gather(0)
    stage_idx(1, 1)
    wait_idx(1)
    start_gather(1)
    stage_idx(2, 2)
    chunk(0, 0, True, False)
    chunk(1, 1, True, False)
    chunk(2, 2, False, False)
    chunk(3, 3, False, False)

    def loop_body(j, carry):
        for r in range(4):
            chunk(4 * j + r, r, False, True)
        return carry

    lax.fori_loop(1, NCH // 4, loop_body, 0)

    wait_scatter((NCH - 2) % 4, (NCH - 2) % 2)
    wait_scatter((NCH - 1) % 4, (NCH - 1) % 2)

    plsc.subcore_barrier()
    pltpu.sync_copy(acc.at[pl.ds(sid * ROWS_PT, ROWS_PT)],
                    out.at[cid, pl.ds(sid * ROWS_PT, ROWS_PT)])


def _scatter_partials(tab, srch, dsth, wh, zeros_hbm):
    mesh = plsc.VectorSubcoreMesh(core_axis_name="c", subcore_axis_name="s")
    return pl.kernel(
        _sc_body,
        mesh=mesh,
        out_type=jax.ShapeDtypeStruct((NC, NPAD, D), jnp.float32),
        scratch_types=[
            pltpu.VMEM_SHARED((NPAD, D), jnp.float32),
            pltpu.VMEM((CH, D), jnp.float32),
            pltpu.VMEM((CH, D), jnp.float32),
            pltpu.VMEM((CH, D), jnp.float32),
            pltpu.VMEM((CH, D), jnp.float32),
            pltpu.VMEM((CH,), jnp.int32),
            pltpu.VMEM((CH,), jnp.int32),
            pltpu.VMEM((CH,), jnp.int32),
            pltpu.VMEM((CH,), jnp.int32),
            pltpu.VMEM((CH,), jnp.int32),
            pltpu.VMEM((CH,), jnp.int32),
            pltpu.VMEM((CH,), jnp.int32),
            pltpu.VMEM((CH,), jnp.int32),
            pltpu.VMEM((CH,), jnp.float32),
            pltpu.VMEM((CH,), jnp.float32),
            pltpu.VMEM((CH,), jnp.float32),
            pltpu.VMEM((CH,), jnp.float32),
            pltpu.VMEM((CH,), jnp.int32),
            pltpu.VMEM((CH,), jnp.int32),
            pltpu.SemaphoreType.DMA,
            pltpu.SemaphoreType.DMA,
            pltpu.SemaphoreType.DMA,
            pltpu.SemaphoreType.DMA,
            pltpu.SemaphoreType.DMA,
        ],
    )(tab, srch, dsth, wh, zeros_hbm)


BM = 2000


def _mm_body(x_ref, y_ref, w1_ref, w2_ref, o_ref):
    dn = (((1,), (1,)), ((), ()))
    s = pl.program_id(0)

    @pl.when(s == 0)
    def _():
        o_ref[...] = lax.dot_general(x_ref[...], w1_ref[...], dn,
                                     preferred_element_type=jnp.float32)[None]

    @pl.when(s == 1)
    def _():
        o_ref[...] = lax.dot_general(y_ref[...], w2_ref[...], dn,
                                     preferred_element_type=jnp.float32)[None]


def _add_body(a_ref, b_ref, o_ref):
    o_ref[...] = a_ref[...] + b_ref[...]


def kernel(sup_x, y, assign_index, assign_weight, anchor_links, anchor_weight,
           num_nodes, W1, W2):
    srca = assign_index[0].astype(jnp.int32)
    dsta = assign_index[1].astype(jnp.int32)
    srcb = anchor_links[0].astype(jnp.int32) + NPAD
    dstb = anchor_links[1].astype(jnp.int32)

    npad_e = E_TOT - 2 * N_EDGES
    pad_idx = (jnp.arange(npad_e, dtype=jnp.int32) % N_NODES)
    src_all = jnp.concatenate([srca, srcb, pad_idx])
    dst_all = jnp.concatenate([dsta, dstb, pad_idx])
    w_all = jnp.concatenate(
        [assign_weight, anchor_weight, jnp.zeros((npad_e,), jnp.float32)])

    npb = N_NODES // BM
    tab = pl.pallas_call(
        _mm_body,
        grid=(2, npb),
        in_specs=[
            pl.BlockSpec((BM, D), lambda s, i: (i, 0)),
            pl.BlockSpec((BM, D), lambda s, i: (i, 0)),
            pl.BlockSpec((D, D), lambda s, i: (0, 0)),
            pl.BlockSpec((D, D), lambda s, i: (0, 0)),
        ],
        out_specs=pl.BlockSpec((1, BM, D), lambda s, i: (s, i, 0)),
        out_shape=jax.ShapeDtypeStruct((2, NPAD, D), jnp.float32),
    )(sup_x, y, W1, W2)

    zeros_hbm = jnp.zeros((NPAD, D), jnp.float32)
    partial = _scatter_partials(tab.reshape(2 * NPAD, D),
                                src_all, dst_all, w_all, zeros_hbm)

    z = pl.pallas_call(
        _add_body,
        grid=(N_NODES // 2000,),
        in_specs=[
            pl.BlockSpec((2000, D), lambda i: (i, 0)),
            pl.BlockSpec((2000, D), lambda i: (i, 0)),
        ],
        out_specs=pl.BlockSpec((2000, D), lambda i: (i, 0)),
        out_shape=jax.ShapeDtypeStruct((N_NODES, D), jnp.float32),
    )(partial[0, :N_NODES], partial[1, :N_NODES])
    return z


# speedup vs baseline: 1.0024x; 1.0024x over previous
import jax
import jax.numpy as jnp
from jax import lax
from jax.experimental import pallas as pl
from jax.experimental.pallas import tpu as pltpu
from jax.experimental.pallas import tpu_sc as plsc

D = 128
N_NODES = 10000
N_EDGES = 320000

NC = 2
NS = 16
NW = NC * NS

NPAD = 10240
ROWS_PT = NPAD // NS

CH = 80
NCH = 252
EPT = NCH * CH
E_TOT = NW * EPT


def _lane_bcast(vec, j):
    idx = jnp.full((16, 1), j, dtype=jnp.int32)
    return lax.gather(
        vec, idx,
        lax.GatherDimensionNumbers(
            offset_dims=(), collapsed_slice_dims=(0,), start_index_map=(0,)),
        (1,),
        mode=lax.GatherScatterMode.PROMISE_IN_BOUNDS)


def _sc_body(tab, srch, dsth, wh, zeros_hbm, out,
             acc, rows0, rows1, rows2, rows3,
             srcs0, srcs1, srcs2, srcs3, dsts0, dsts1, dsts2, dsts3,
             ws0, ws1, ws2, ws3, dsc0, dsc1, g0, g1, ssem, isem):
    cid = lax.axis_index("c")
    sid = lax.axis_index("s")
    wid = sid * NC + cid

    rows = (rows0, rows1, rows2, rows3)
    srcs = (srcs0, srcs1, srcs2, srcs3)
    dsts = (dsts0, dsts1, dsts2, dsts3)
    ws = (ws0, ws1, ws2, ws3)
    dsc = (dsc0, dsc1)
    gsem = (g0, g1)

    pltpu.sync_copy(zeros_hbm.at[pl.ds(sid * ROWS_PT, ROWS_PT)],
                    acc.at[pl.ds(sid * ROWS_PT, ROWS_PT)])
    plsc.subcore_barrier()

    base = wid * EPT

    def stage_idx(c, s):
        off = base + c * CH
        pltpu.async_copy(srch.at[pl.ds(off, CH)], srcs[s], isem)
        pltpu.async_copy(dsth.at[pl.ds(off, CH)], dsts[s], isem)
        pltpu.async_copy(wh.at[pl.ds(off, CH)], ws[s], isem)

    def wait_idx(s):
        pltpu.make_async_copy(srch.at[pl.ds(0, CH)], srcs[s], isem).wait()
        pltpu.make_async_copy(dsth.at[pl.ds(0, CH)], dsts[s], isem).wait()
        pltpu.make_async_copy(wh.at[pl.ds(0, CH)], ws[s], isem).wait()

    def start_gather(l):
        pltpu.async_copy(tab.at[srcs[l]], rows[l], gsem[l % 2])

    def wait_gather(l):
        pltpu.make_async_copy(tab.at[srcs[l]], rows[l], gsem[l % 2]).wait()

    def start_scatter(l, p):
        for k in range(CH // 16):
            dsc[p][pl.ds(16 * k, 16)] = dsts[l][pl.ds(16 * k, 16)]
        pltpu.async_copy(rows[l], acc.at[dsc[p]], ssem, add=True)

    def wait_scatter(l, p):
        pltpu.make_async_copy(rows[l], acc.at[dsc[p]], ssem).wait()

    def scale(b, s):
        rbuf = rows[b]
        wref = ws[s]

        def grp(g, carry):
            wv = wref[pl.ds(16 * g, 16)]
            for j in range(16):
                wbc = _lane_bcast(wv, j)
                e = 16 * g + j
                for v in range(8):
                    rbuf[e, pl.ds(16 * v, 16)] = rbuf[e, pl.ds(16 * v, 16)] * wbc
            return carry
        lax.fori_loop(0, CH // 16, grp, 0)

    def chunk(c, r, first, guard):
        l = r
        l2 = (r + 2) % 4
        l3 = (r + 3) % 4
        wait_gather(l)
        if guard:
            @pl.when(c + 2 < NCH)
            def _():
                wait_idx(l2)

            @pl.when(c + 3 < NCH)
            def _():
                stage_idx(c + 3, l3)

            @pl.when(c + 2 < NCH)
            def _():
                start_gather(l2)
        else:
            wait_idx(l2)
            stage_idx(c + 3, l3)
            start_gather(l2)
        scale(l, l)
        if not first:
            wait_scatter(l3, (r + 1) % 2)
        start_scatter(l, r % 2)

    stage_idx(0, 0)
    wait_idx(0)
    start_gather(0)
    stage_idx(1, 1)
    wait_idx(1)
    start_gather(1)
    stage_idx(2, 2)
    chunk(0, 0, True, False)
    chunk(1, 1, False, False)
    chunk(2, 2, False, False)
    chunk(3, 3, False, False)

    def loop_body(j, carry):
        for r in range(4):
            chunk(4 * j + r, r, False, True)
        return carry

    lax.fori_loop(1, NCH // 4, loop_body, 0)

    wait_scatter((NCH - 1) % 4, (NCH - 1) % 2)

    plsc.subcore_barrier()
    pltpu.sync_copy(acc.at[pl.ds(sid * ROWS_PT, ROWS_PT)],
                    out.at[cid, pl.ds(sid * ROWS_PT, ROWS_PT)])


def _scatter_partials(tab, srch, dsth, wh, zeros_hbm):
    mesh = plsc.VectorSubcoreMesh(core_axis_name="c", subcore_axis_name="s")
    return pl.kernel(
        _sc_body,
        mesh=mesh,
        out_type=jax.ShapeDtypeStruct((NC, NPAD, D), jnp.float32),
        scratch_types=[
            pltpu.VMEM_SHARED((NPAD, D), jnp.float32),
            pltpu.VMEM((CH, D), jnp.float32),
            pltpu.VMEM((CH, D), jnp.float32),
            pltpu.VMEM((CH, D), jnp.float32),
            pltpu.VMEM((CH, D), jnp.float32),
            pltpu.VMEM((CH,), jnp.int32),
            pltpu.VMEM((CH,), jnp.int32),
            pltpu.VMEM((CH,), jnp.int32),
            pltpu.VMEM((CH,), jnp.int32),
            pltpu.VMEM((CH,), jnp.int32),
            pltpu.VMEM((CH,), jnp.int32),
            pltpu.VMEM((CH,), jnp.int32),
            pltpu.VMEM((CH,), jnp.int32),
            pltpu.VMEM((CH,), jnp.float32),
            pltpu.VMEM((CH,), jnp.float32),
            pltpu.VMEM((CH,), jnp.float32),
            pltpu.VMEM((CH,), jnp.float32),
            pltpu.VMEM((CH,), jnp.int32),
            pltpu.VMEM((CH,), jnp.int32),
            pltpu.SemaphoreType.DMA,
            pltpu.SemaphoreType.DMA,
            pltpu.SemaphoreType.DMA,
            pltpu.SemaphoreType.DMA,
        ],
    )(tab, srch, dsth, wh, zeros_hbm)


BM = 2000


def _mm_body(x_ref, y_ref, w1_ref, w2_ref, o_ref):
    dn = (((1,), (1,)), ((), ()))
    s = pl.program_id(0)

    @pl.when(s == 0)
    def _():
        o_ref[...] = lax.dot_general(x_ref[...], w1_ref[...], dn,
                                     preferred_element_type=jnp.float32)[None]

    @pl.when(s == 1)
    def _():
        o_ref[...] = lax.dot_general(y_ref[...], w2_ref[...], dn,
                                     preferred_element_type=jnp.float32)[None]


def _add_body(a_ref, b_ref, o_ref):
    o_ref[...] = a_ref[...] + b_ref[...]


def kernel(sup_x, y, assign_index, assign_weight, anchor_links, anchor_weight,
           num_nodes, W1, W2):
    srca = assign_index[0].astype(jnp.int32)
    dsta = assign_index[1].astype(jnp.int32)
    srcb = anchor_links[0].astype(jnp.int32) + NPAD
    dstb = anchor_links[1].astype(jnp.int32)

    npad_e = E_TOT - 2 * N_EDGES
    pad_idx = (jnp.arange(npad_e, dtype=jnp.int32) % N_NODES)
    src_all = jnp.concatenate([srca, srcb, pad_idx])
    dst_all = jnp.concatenate([dsta, dstb, pad_idx])
    w_all = jnp.concatenate(
        [assign_weight, anchor_weight, jnp.zeros((npad_e,), jnp.float32)])

    npb = N_NODES // BM
    tab = pl.pallas_call(
        _mm_body,
        grid=(2, npb),
        in_specs=[
            pl.BlockSpec((BM, D), lambda s, i: (i, 0)),
            pl.BlockSpec((BM, D), lambda s, i: (i, 0)),
            pl.BlockSpec((D, D), lambda s, i: (0, 0)),
            pl.BlockSpec((D, D), lambda s, i: (0, 0)),
        ],
        out_specs=pl.BlockSpec((1, BM, D), lambda s, i: (s, i, 0)),
        out_shape=jax.ShapeDtypeStruct((2, NPAD, D), jnp.float32),
    )(sup_x, y, W1, W2)

    zeros_hbm = jnp.zeros((NPAD, D), jnp.float32)
    partial = _scatter_partials(tab.reshape(2 * NPAD, D),
                                src_all, dst_all, w_all, zeros_hbm)

    z = pl.pallas_call(
        _add_body,
        grid=(N_NODES // 2000,),
        in_specs=[
            pl.BlockSpec((2000, D), lambda i: (i, 0)),
            pl.BlockSpec((2000, D), lambda i: (i, 0)),
        ],
        out_specs=pl.BlockSpec((2000, D), lambda i: (i, 0)),
        out_shape=jax.ShapeDtypeStruct((N_NODES, D), jnp.float32),
    )(partial[0, :N_NODES], partial[1, :N_NODES])
    return z
